# Initial kernel scaffold; baseline (speedup 1.0000x reference)
import jax
import jax.numpy as jnp
from jax.experimental import pallas as pl


def kernel(query, key, rel_bias):
    raise NotImplementedError("write your pallas kernel here")


# speedup vs baseline: 53.5316x; 53.5316x over previous
import functools
import math

import jax
import jax.numpy as jnp
from jax import lax
from jax.experimental import pallas as pl
from jax.experimental.pallas import tpu as pltpu
from jax.experimental.pallas import tpu_sc as plsc

NUM_HEADS = 12
NUM_BUCKETS = 32
MAX_DISTANCE = 128
QLEN = 2048
KLEN = 2048
SHIFTS = 16
DIAG_LANES = 4080
NUM_WORKERS = 32
ROWS_PER_WORKER = (NUM_HEADS * QLEN) // NUM_WORKERS
BATCH = 16
NBATCH = ROWS_PER_WORKER // BATCH


def _diag_table_kernel(rel_bias_ref, out_ref):
    s = lax.broadcasted_iota(jnp.int32, (SHIFTS, DIAG_LANES), 0)
    u = lax.broadcasted_iota(jnp.int32, (SHIFTS, DIAG_LANES), 1)
    t = u + s
    n = (QLEN - 1) - t
    half = NUM_BUCKETS // 2
    max_exact = half // 2
    ret = jnp.where(n < 0, half, 0)
    na = jnp.abs(n)
    is_small = na < max_exact
    nf = jnp.maximum(na.astype(jnp.float32), 1.0) / max_exact
    val_if_large = max_exact + (
        jnp.log(nf) / math.log(MAX_DISTANCE / max_exact) * (half - max_exact)
    ).astype(jnp.int32)
    val_if_large = jnp.minimum(val_if_large, half - 1)
    bucket = ret + jnp.where(is_small, na, val_if_large)
    for h in range(NUM_HEADS):
        acc = jnp.zeros((SHIFTS, DIAG_LANES), jnp.float32)
        for b in range(NUM_BUCKETS):
            acc = jnp.where(bucket == b, rel_bias_ref[b, h], acc)
        out_ref[h] = acc


_HEAD_WORDS = SHIFTS * DIAG_LANES


def _fanout_body(diag_hbm, out_hbm, diag_v, sem):
    wid = lax.axis_index("s") * 2 + lax.axis_index("c")
    lo = wid * ROWS_PER_WORKER
    h_lo = lo // QLEN
    h_hi = (lo + ROWS_PER_WORKER - 1) // QLEN
    pltpu.sync_copy(
        diag_hbm.at[pl.ds(h_lo * _HEAD_WORDS, _HEAD_WORDS)],
        diag_v.at[pl.ds(0, _HEAD_WORDS)],
    )
    pltpu.sync_copy(
        diag_hbm.at[pl.ds(h_hi * _HEAD_WORDS, _HEAD_WORDS)],
        diag_v.at[pl.ds(_HEAD_WORDS, _HEAD_WORDS)],
    )

    def batch_body(g, carry):
        rbase = lo + g * BATCH
        copies = []
        for j in range(BATCH):
            r = rbase + j
            h = r >> 11
            i = r & (QLEN - 1)
            start = (QLEN - 1) - i
            s = start & (SHIFTS - 1)
            base = start - s
            src_off = pl.multiple_of(
                (h - h_lo) * _HEAD_WORDS + s * DIAG_LANES + base, SHIFTS
            )
            cp = pltpu.make_async_copy(
                diag_v.at[pl.ds(src_off, KLEN)],
                out_hbm.at[pl.ds(r * KLEN, KLEN)],
                sem,
            )
            cp.start()
            copies.append(cp)
        for cp in copies:
            cp.wait()
        return carry

    lax.fori_loop(0, NBATCH, batch_body, 0)


def kernel(query, key, rel_bias):
    batch_size = query.shape[0]

    diag16 = pl.pallas_call(
        _diag_table_kernel,
        out_shape=jax.ShapeDtypeStruct((NUM_HEADS, SHIFTS, DIAG_LANES), jnp.float32),
    )(rel_bias)

    fanout = pl.kernel(
        _fanout_body,
        out_type=jax.ShapeDtypeStruct((NUM_HEADS * QLEN * KLEN,), jnp.float32),
        mesh=plsc.VectorSubcoreMesh(core_axis_name="c", subcore_axis_name="s"),
        scratch_types=[
            pltpu.VMEM((2 * _HEAD_WORDS,), jnp.float32),
            pltpu.SemaphoreType.DMA,
        ],
    )
    out_flat = fanout(diag16.reshape(NUM_HEADS * _HEAD_WORDS))
    out = out_flat.reshape(1, NUM_HEADS, QLEN, KLEN)
    return jnp.broadcast_to(out, (batch_size, NUM_HEADS, QLEN, KLEN))
